# baseline (speedup 1.0000x reference)
import functools

import jax
import jax.numpy as jnp
from jax import lax
from jax.experimental import pallas as pl
from jax.experimental.pallas import tpu as pltpu
from jax.experimental.pallas import tpu_sc as plsc

NC = 2
NS = 16
K = 128
BR = 256


def _make_deg_kernel(NPAD, C, interpret=False):
  rpt = NPAD // NS
  W = 8

  @functools.partial(
      pl.kernel,
      out_type=jax.ShapeDtypeStruct((NC, NPAD, 8), jnp.float32),
      mesh=plsc.VectorSubcoreMesh(core_axis_name="c", subcore_axis_name="s",
                                  num_cores=NC, num_subcores=NS),
      scratch_types=[
          pltpu.VMEM_SHARED((NPAD, 8), jnp.float32),
          pltpu.VMEM((C, 2, K), jnp.int32),
          pltpu.VMEM((K, 8), jnp.float32),
          pltpu.SemaphoreType.DMA,
      ],
      interpret=interpret,
  )
  def deg_kernel(eidx_hbm, ones_hbm, zeros_hbm, out_hbm, acc, didx, ones_v,
                 sem):
    c = lax.axis_index("c")
    s = lax.axis_index("s")
    pltpu.sync_copy(zeros_hbm, acc.at[pl.ds(s * rpt, rpt)])
    pltpu.sync_copy(ones_hbm, ones_v)
    row0 = (c * NS + s) * C
    pltpu.sync_copy(eidx_hbm.at[pl.ds(row0, C)], didx)
    plsc.subcore_barrier()

    def drain_one():
      pltpu.make_async_copy(ones_v, acc.at[didx.at[0, 1]], sem).wait()

    def step(i, carry):
      pltpu.async_copy(ones_v, acc.at[didx.at[i, 1]], sem, add=True)

      @pl.when(i >= W)
      def _():
        drain_one()

      return carry

    lax.fori_loop(0, C, step, 0)
    for _ in range(W):
      drain_one()
    plsc.subcore_barrier()
    pltpu.sync_copy(acc.at[pl.ds(s * rpt, rpt)],
                    out_hbm.at[c, pl.ds(s * rpt, rpt)])

  return deg_kernel


def _make_agg_kernel(NPAD, D, C, interpret=False):
  rpt = NPAD // NS
  assert C % 4 == 0

  @functools.partial(
      pl.kernel,
      out_type=jax.ShapeDtypeStruct((NC, NPAD, D), jnp.float32),
      mesh=plsc.VectorSubcoreMesh(core_axis_name="c", subcore_axis_name="s",
                                  num_cores=NC, num_subcores=NS),
      scratch_types=[
          pltpu.VMEM_SHARED((NPAD, D), jnp.float32),
          pltpu.VMEM((4, 2, K), jnp.int32),
          pltpu.VMEM((2, K, D), jnp.float32),
          pltpu.SemaphoreType.DMA,
          pltpu.SemaphoreType.DMA,
          pltpu.SemaphoreType.DMA,
          pltpu.SemaphoreType.DMA,
          pltpu.SemaphoreType.DMA,
      ],
      interpret=interpret,
  )
  def agg_kernel(g_hbm, eidx_hbm, zeros_hbm, out_hbm, acc, idx, rows,
                 sem_g0, sem_g1, sem_s0, sem_s1, sem_i):
    c = lax.axis_index("c")
    s = lax.axis_index("s")
    sem_g = (sem_g0, sem_g1)
    sem_s = (sem_s0, sem_s1)
    pltpu.sync_copy(zeros_hbm, acc.at[pl.ds(s * rpt, rpt)])
    row0 = (c * NS + s) * C

    def issue_idx(i, q):
      pltpu.async_copy(eidx_hbm.at[row0 + i], idx.at[q], sem_i)

    def wait_idx():
      pltpu.make_async_copy(eidx_hbm.at[row0], idx.at[0], sem_i).wait()

    def issue_gather(q, p):
      pltpu.async_copy(g_hbm.at[idx.at[q, 0]], rows.at[p], sem_g[p])

    def wait_gather(p):
      pltpu.make_async_copy(g_hbm.at[idx.at[0, 0]], rows.at[p],
                            sem_g[p]).wait()

    def issue_scatter(q, p):
      pltpu.async_copy(rows.at[p], acc.at[idx.at[q, 1]], sem_s[p], add=True)

    def wait_scatter(p):
      pltpu.make_async_copy(rows.at[p], acc.at[idx.at[0, 1]],
                            sem_s[p]).wait()

    issue_idx(0, 0)
    plsc.subcore_barrier()

    @pl.loop(0, C, step=4)
    def _(gg):
      for po in range(4):
        i = gg + po
        p = po % 2

        @pl.when(i >= 2)
        def _():
          wait_scatter(p)
        wait_idx()
        issue_gather(po, p)

        @pl.when(i + 1 < C)
        def _():
          issue_idx(i + 1, (po + 1) % 4)

        @pl.when(i >= 1)
        def _():
          wait_gather(1 - p)
          issue_scatter((po - 1) % 4, 1 - p)

    wait_gather(1)
    issue_scatter(3, 1)
    wait_scatter(0)
    wait_scatter(1)
    plsc.subcore_barrier()
    pltpu.sync_copy(acc.at[pl.ds(s * rpt, rpt)],
                    out_hbm.at[c, pl.ds(s * rpt, rpt)])

  return agg_kernel


def _mm_body(x_ref, w_ref, o_ref):
  o_ref[...] = jnp.dot(x_ref[...], w_ref[...],
                       preferred_element_type=jnp.float32)


def _scale_body(h_ref, deg_ref, g_ref, dinv_ref):
  deg = deg_ref[0] + deg_ref[1] + 1.0
  dinv = lax.rsqrt(deg)
  dinv_ref[...] = dinv
  g_ref[...] = h_ref[...] * dinv[:, :1]


def _layer_body(p_ref, g_ref, dinv_ref, b_ref, w_ref, o_ref):
  dinv = dinv_ref[:, :1]
  t = dinv * (p_ref[0] + p_ref[1] + g_ref[...]) + b_ref[...]
  t = jnp.maximum(t, 0.0)
  o_ref[...] = dinv * jnp.dot(t, w_ref[...],
                              preferred_element_type=jnp.float32)


def _final_body(q_ref, g_ref, dinv_ref, b_ref, o_ref):
  dinv = dinv_ref[:, :1]
  t = dinv * (q_ref[0] + q_ref[1] + g_ref[...]) + b_ref[...]
  o_ref[...] = jnp.maximum(t, 0.0)


def _gcn(x, edge_index, W1, b1, W2, b2, interpret=False):
  N, D = x.shape
  E = edge_index.shape[1]
  NPAD = ((N + 2047) // 2048) * 2048
  group = NC * NS * K * 4
  C = (-(-E // group)) * 4
  EPAD = NC * NS * K * C

  xp = jnp.zeros((NPAD, D), jnp.float32).at[:N].set(x)
  pad = jnp.full((EPAD - E,), N, jnp.int32)
  srcp = jnp.concatenate([edge_index[0], pad]).reshape(-1, 1, K)
  dstp = jnp.concatenate([edge_index[1], pad]).reshape(-1, 1, K)
  eidx = jnp.concatenate([srcp, dstp], axis=1)

  ones8 = jnp.ones((K, 8), jnp.float32)
  zeros8 = jnp.zeros((NPAD // NS, 8), jnp.float32)
  zerosD = jnp.zeros((NPAD // NS, D), jnp.float32)

  deg8 = _make_deg_kernel(NPAD, C, interpret)(eidx, ones8, zeros8)

  grid = (NPAD // BR,)
  blk = pl.BlockSpec((BR, D), lambda i: (i, 0))
  blk8 = pl.BlockSpec((BR, 8), lambda i: (i, 0))
  blk2 = pl.BlockSpec((NC, BR, D), lambda i: (0, i, 0))
  blk28 = pl.BlockSpec((NC, BR, 8), lambda i: (0, i, 0))
  blkw = pl.BlockSpec((D, D), lambda i: (0, 0))
  blkb = pl.BlockSpec((1, D), lambda i: (0, 0))
  fD = jax.ShapeDtypeStruct((NPAD, D), jnp.float32)
  f8 = jax.ShapeDtypeStruct((NPAD, 8), jnp.float32)

  h1 = pl.pallas_call(
      _mm_body, grid=grid, in_specs=[blk, blkw], out_specs=blk,
      out_shape=fD, interpret=interpret)(xp, W1)

  g1, dinv8 = pl.pallas_call(
      _scale_body, grid=grid, in_specs=[blk, blk28],
      out_specs=[blk, blk8], out_shape=[fD, f8],
      interpret=interpret)(h1, deg8)

  agg = _make_agg_kernel(NPAD, D, C, interpret)
  P = agg(g1, eidx, zerosD)

  b1r = b1.reshape(1, D)
  b2r = b2.reshape(1, D)
  g2 = pl.pallas_call(
      _layer_body, grid=grid, in_specs=[blk2, blk, blk8, blkb, blkw],
      out_specs=blk, out_shape=fD, interpret=interpret)(
          P, g1, dinv8, b1r, W2)

  Q = agg(g2, eidx, zerosD)

  out = pl.pallas_call(
      _final_body, grid=grid, in_specs=[blk2, blk, blk8, blkb],
      out_specs=blk, out_shape=fD, interpret=interpret)(
          Q, g2, dinv8, b2r)

  return out[:N]


def kernel(x, edge_index, W1, b1, W2, b2):
  return _gcn(x, edge_index, W1, b1, W2, b2)


# speedup vs baseline: 1.0176x; 1.0176x over previous
import functools

import jax
import jax.numpy as jnp
from jax import lax
from jax.experimental import pallas as pl
from jax.experimental.pallas import tpu as pltpu
from jax.experimental.pallas import tpu_sc as plsc

NC = 2
NS = 16
K = 128
BR = 256


def _split(NSC):
  rpt = -(-NSC // NS // 8) * 8
  last = NSC - (NS - 1) * rpt
  assert last > 0 and last % 8 == 0
  return rpt, last


def _make_deg_kernel(NPAD, NSC, C, interpret=False):
  rpt, last = _split(NSC)
  W = 8

  @functools.partial(
      pl.kernel,
      out_type=jax.ShapeDtypeStruct((NC, NPAD, 8), jnp.float32),
      mesh=plsc.VectorSubcoreMesh(core_axis_name="c", subcore_axis_name="s",
                                  num_cores=NC, num_subcores=NS),
      scratch_types=[
          pltpu.VMEM_SHARED((NSC, 8), jnp.float32),
          pltpu.VMEM((C, 2, K), jnp.int32),
          pltpu.VMEM((K, 8), jnp.float32),
          pltpu.SemaphoreType.DMA,
      ],
      interpret=interpret,
  )
  def deg_kernel(eidx_hbm, ones_hbm, zeros_hbm, out_hbm, acc, didx, ones_v,
                 sem):
    c = lax.axis_index("c")
    s = lax.axis_index("s")

    @pl.when(s < NS - 1)
    def _():
      pltpu.sync_copy(zeros_hbm, acc.at[pl.ds(s * rpt, rpt)])

    @pl.when(s == NS - 1)
    def _():
      pltpu.sync_copy(zeros_hbm.at[pl.ds(0, last)],
                      acc.at[pl.ds((NS - 1) * rpt, last)])

    pltpu.sync_copy(ones_hbm, ones_v)
    row0 = (c * NS + s) * C
    pltpu.sync_copy(eidx_hbm.at[pl.ds(row0, C)], didx)
    plsc.subcore_barrier()

    def drain_one():
      pltpu.make_async_copy(ones_v, acc.at[didx.at[0, 1]], sem).wait()

    def step(i, carry):
      pltpu.async_copy(ones_v, acc.at[didx.at[i, 1]], sem, add=True)

      @pl.when(i >= W)
      def _():
        drain_one()

      return carry

    lax.fori_loop(0, C, step, 0)
    for _ in range(W):
      drain_one()
    plsc.subcore_barrier()

    @pl.when(s < NS - 1)
    def _():
      pltpu.sync_copy(acc.at[pl.ds(s * rpt, rpt)],
                      out_hbm.at[c, pl.ds(s * rpt, rpt)])

    @pl.when(s == NS - 1)
    def _():
      pltpu.sync_copy(acc.at[pl.ds((NS - 1) * rpt, last)],
                      out_hbm.at[c, pl.ds((NS - 1) * rpt, last)])

  return deg_kernel


def _make_agg_kernel(NPAD, NSC, D, C, interpret=False):
  rpt, last = _split(NSC)
  assert C % 6 == 2 and C >= 14

  @functools.partial(
      pl.kernel,
      out_type=jax.ShapeDtypeStruct((NC, NPAD, D), jnp.float32),
      mesh=plsc.VectorSubcoreMesh(core_axis_name="c", subcore_axis_name="s",
                                  num_cores=NC, num_subcores=NS),
      scratch_types=[
          pltpu.VMEM_SHARED((NSC, D), jnp.float32),
          pltpu.VMEM((6, 2, K), jnp.int32),
          pltpu.VMEM((3, K, D), jnp.float32),
          pltpu.SemaphoreType.DMA,
          pltpu.SemaphoreType.DMA,
          pltpu.SemaphoreType.DMA,
          pltpu.SemaphoreType.DMA,
          pltpu.SemaphoreType.DMA,
          pltpu.SemaphoreType.DMA,
          pltpu.SemaphoreType.DMA,
          pltpu.SemaphoreType.DMA,
          pltpu.SemaphoreType.DMA,
          pltpu.SemaphoreType.DMA,
          pltpu.SemaphoreType.DMA,
          pltpu.SemaphoreType.DMA,
      ],
      interpret=interpret,
  )
  def agg_kernel(g_hbm, eidx_hbm, zeros_hbm, out_hbm, acc, idx, rows,
                 sg0, sg1, sg2, ss0, ss1, ss2,
                 si0, si1, si2, si3, si4, si5):
    c = lax.axis_index("c")
    s = lax.axis_index("s")
    sem_g = (sg0, sg1, sg2)
    sem_s = (ss0, ss1, ss2)
    sem_i = (si0, si1, si2, si3, si4, si5)
    row0 = (c * NS + s) * C

    def issue_idx(i, q):
      pltpu.async_copy(eidx_hbm.at[row0 + i], idx.at[q], sem_i[q])

    def wait_idx(q):
      pltpu.make_async_copy(eidx_hbm.at[row0], idx.at[q], sem_i[q]).wait()

    def issue_gather(q, p):
      pltpu.async_copy(g_hbm.at[idx.at[q, 0]], rows.at[p], sem_g[p])

    def wait_gather(p):
      pltpu.make_async_copy(g_hbm.at[idx.at[0, 0]], rows.at[p],
                            sem_g[p]).wait()

    def issue_scatter(q, p):
      pltpu.async_copy(rows.at[p], acc.at[idx.at[q, 1]], sem_s[p], add=True)

    def wait_scatter(p):
      pltpu.make_async_copy(rows.at[p], acc.at[idx.at[0, 1]],
                            sem_s[p]).wait()

    @pl.when(s < NS - 1)
    def _():
      pltpu.sync_copy(zeros_hbm, acc.at[pl.ds(s * rpt, rpt)])

    @pl.when(s == NS - 1)
    def _():
      pltpu.sync_copy(zeros_hbm.at[pl.ds(0, last)],
                      acc.at[pl.ds((NS - 1) * rpt, last)])

    for q in range(4):
      issue_idx(q, q)
    plsc.subcore_barrier()
    wait_idx(0)
    issue_gather(0, 0)

    @pl.loop(0, C - 2, step=6)
    def _(gg):
      for po in range(6):
        i = gg + po

        @pl.when(i + 1 < C)
        def _():
          @pl.when(i >= 2)
          def _():
            wait_scatter((po + 1) % 3)
          wait_idx((po + 1) % 6)
          issue_gather((po + 1) % 6, (po + 1) % 3)

        @pl.when(i + 4 < C)
        def _():
          issue_idx(i + 4, (po + 4) % 6)

        wait_gather(po % 3)
        issue_scatter(po, po % 3)

    wait_scatter(1)
    wait_idx(1)
    issue_gather(1, 1)
    wait_gather(0)
    issue_scatter(0, 0)
    wait_gather(1)
    issue_scatter(1, 1)
    for p in (2, 0, 1):
      wait_scatter(p)
    plsc.subcore_barrier()

    @pl.when(s < NS - 1)
    def _():
      pltpu.sync_copy(acc.at[pl.ds(s * rpt, rpt)],
                      out_hbm.at[c, pl.ds(s * rpt, rpt)])

    @pl.when(s == NS - 1)
    def _():
      pltpu.sync_copy(acc.at[pl.ds((NS - 1) * rpt, last)],
                      out_hbm.at[c, pl.ds((NS - 1) * rpt, last)])

  return agg_kernel


def _mm_scale_body(x_ref, w_ref, deg_ref, g_ref, dinv_ref):
  deg = deg_ref[0] + deg_ref[1] + 1.0
  dinv = lax.rsqrt(deg)
  dinv_ref[...] = dinv
  h = jnp.dot(x_ref[...], w_ref[...], preferred_element_type=jnp.float32)
  g_ref[...] = h * dinv[:, :1]


def _layer_body(p_ref, g_ref, dinv_ref, b_ref, w_ref, o_ref):
  dinv = dinv_ref[:, :1]
  t = dinv * (p_ref[0] + p_ref[1] + g_ref[...]) + b_ref[...]
  t = jnp.maximum(t, 0.0)
  o_ref[...] = dinv * jnp.dot(t, w_ref[...],
                              preferred_element_type=jnp.float32)


def _final_body(q_ref, g_ref, dinv_ref, b_ref, o_ref):
  dinv = dinv_ref[:, :1]
  t = dinv * (q_ref[0] + q_ref[1] + g_ref[...]) + b_ref[...]
  o_ref[...] = jnp.maximum(t, 0.0)


def _gcn(x, edge_index, W1, b1, W2, b2, interpret=False):
  N, D = x.shape
  E = edge_index.shape[1]
  NPAD = ((N + 2047) // 2048) * 2048
  NSC = ((N + 16) // 16) * 16
  C0 = -(-E // (NC * NS * K))
  C = C0 + ((2 - C0) % 6)
  EPAD = NC * NS * K * C

  xp = jnp.zeros((NPAD, D), jnp.float32).at[:N].set(x)
  pad = jnp.full((EPAD - E,), N, jnp.int32)
  srcp = jnp.concatenate([edge_index[0], pad]).reshape(-1, 1, K)
  dstp = jnp.concatenate([edge_index[1], pad]).reshape(-1, 1, K)
  eidx = jnp.concatenate([srcp, dstp], axis=1)

  rpt, _ = _split(NSC)
  ones8 = jnp.ones((K, 8), jnp.float32)
  zeros8 = jnp.zeros((rpt, 8), jnp.float32)
  zerosD = jnp.zeros((rpt, D), jnp.float32)

  deg8 = _make_deg_kernel(NPAD, NSC, C, interpret)(eidx, ones8, zeros8)

  grid = (NPAD // BR,)
  blk = pl.BlockSpec((BR, D), lambda i: (i, 0))
  blk8 = pl.BlockSpec((BR, 8), lambda i: (i, 0))
  blk2 = pl.BlockSpec((NC, BR, D), lambda i: (0, i, 0))
  blk28 = pl.BlockSpec((NC, BR, 8), lambda i: (0, i, 0))
  blkw = pl.BlockSpec((D, D), lambda i: (0, 0))
  blkb = pl.BlockSpec((1, D), lambda i: (0, 0))
  fD = jax.ShapeDtypeStruct((NPAD, D), jnp.float32)
  f8 = jax.ShapeDtypeStruct((NPAD, 8), jnp.float32)

  g1, dinv8 = pl.pallas_call(
      _mm_scale_body, grid=grid, in_specs=[blk, blkw, blk28],
      out_specs=[blk, blk8], out_shape=[fD, f8],
      interpret=interpret)(xp, W1, deg8)

  agg = _make_agg_kernel(NPAD, NSC, D, C, interpret)
  P = agg(g1, eidx, zerosD)

  b1r = b1.reshape(1, D)
  b2r = b2.reshape(1, D)
  g2 = pl.pallas_call(
      _layer_body, grid=grid, in_specs=[blk2, blk, blk8, blkb, blkw],
      out_specs=blk, out_shape=fD, interpret=interpret)(
          P, g1, dinv8, b1r, W2)

  Q = agg(g2, eidx, zerosD)

  out = pl.pallas_call(
      _final_body, grid=grid, in_specs=[blk2, blk, blk8, blkb],
      out_specs=blk, out_shape=fD, interpret=interpret)(
          Q, g2, dinv8, b2r)

  return out[:N]


def kernel(x, edge_index, W1, b1, W2, b2):
  return _gcn(x, edge_index, W1, b1, W2, b2)
